# Initial kernel scaffold; baseline (speedup 1.0000x reference)
import jax
import jax.numpy as jnp
from jax.experimental import pallas as pl


def kernel(x, values):
    raise NotImplementedError("write your pallas kernel here")


# speedup vs baseline: 1.4798x; 1.4798x over previous
import functools

import jax
import jax.numpy as jnp
from jax import lax
from jax.experimental import pallas as pl
from jax.experimental.pallas import tpu as pltpu
from jax.experimental.pallas import tpu_sc as plsc

NUM_KNOTS = 64
LANES = 16
NUM_CORES = 2
NUM_SUBCORES = 16
NW = NUM_CORES * NUM_SUBCORES

N_TOTAL = 16384 * 200
PER_W = N_TOTAL // NW
BLK = 4096
NBLK = PER_W // BLK


def _spline_body(a_hbm, b_hbm, c_hbm, d_hbm, x_hbm, out_hbm,
                 a_v, b_v, c_v, d_v, xbuf, obuf):
    wid = lax.axis_index("s") * NUM_CORES + lax.axis_index("c")
    base = wid * PER_W

    pltpu.sync_copy(a_hbm, a_v)
    pltpu.sync_copy(b_hbm, b_v)
    pltpu.sync_copy(c_hbm, c_v)
    pltpu.sync_copy(d_hbm, d_v)

    scale = jnp.float32(NUM_KNOTS - 1)

    def block(blk, _):
        off = base + blk * BLK
        pltpu.sync_copy(x_hbm.at[pl.ds(off, BLK)], xbuf)

        def vec(v, _):
            xv = xbuf[pl.ds(v * LANES, LANES)]
            xv = jnp.minimum(jnp.maximum(xv, jnp.float32(0.0)), jnp.float32(1.0))
            tf = xv * scale
            i = jnp.minimum(tf.astype(jnp.int32), NUM_KNOTS - 2)
            t = tf - i.astype(jnp.float32)
            av = plsc.load_gather(a_v, [i])
            bv = plsc.load_gather(b_v, [i])
            cv = plsc.load_gather(c_v, [i])
            dv = plsc.load_gather(d_v, [i])
            r = ((dv * t + cv) * t + bv) * t + av
            obuf[pl.ds(v * LANES, LANES)] = r
            return _

        lax.fori_loop(0, BLK // LANES, vec, None)
        pltpu.sync_copy(obuf, out_hbm.at[pl.ds(off, BLK)])
        return _

    lax.fori_loop(0, NBLK, block, None)


@functools.partial(jax.jit, static_argnames=())
def kernel(x, values):
    v = values.astype(jnp.float32)
    pm1 = jnp.concatenate([v[:1], v[:-1]])
    pp1 = jnp.concatenate([v[1:], v[-1:]])
    pp2 = jnp.concatenate([v[2:], v[-1:], v[-1:]])
    at = v
    bt = 0.5 * (pp1 - pm1)
    ct = 0.5 * (2.0 * pm1 - 5.0 * v + 4.0 * pp1 - pp2)
    dt = 0.5 * (-pm1 + 3.0 * v - 3.0 * pp1 + pp2)

    xf = x.reshape(-1)

    run = pl.kernel(
        _spline_body,
        out_type=jax.ShapeDtypeStruct((N_TOTAL,), jnp.float32),
        mesh=plsc.VectorSubcoreMesh(
            core_axis_name="c", subcore_axis_name="s",
            num_cores=NUM_CORES, num_subcores=NUM_SUBCORES),
        compiler_params=pltpu.CompilerParams(needs_layout_passes=False),
        scratch_types=[
            pltpu.VMEM((NUM_KNOTS,), jnp.float32),
            pltpu.VMEM((NUM_KNOTS,), jnp.float32),
            pltpu.VMEM((NUM_KNOTS,), jnp.float32),
            pltpu.VMEM((NUM_KNOTS,), jnp.float32),
            pltpu.VMEM((BLK,), jnp.float32),
            pltpu.VMEM((BLK,), jnp.float32),
        ],
    )
    out = run(at, bt, ct, dt, xf)
    return out.reshape(x.shape)


# speedup vs baseline: 2.0856x; 1.4094x over previous
import functools

import jax
import jax.numpy as jnp
from jax import lax
from jax.experimental import pallas as pl
from jax.experimental.pallas import tpu as pltpu
from jax.experimental.pallas import tpu_sc as plsc

NUM_KNOTS = 64
LANES = 16
NUM_CORES = 2
NUM_SUBCORES = 16
NW = NUM_CORES * NUM_SUBCORES

N_TOTAL = 16384 * 200
PER_W = N_TOTAL // NW
BLK = 6400
NBLK = PER_W // BLK


def _spline_body(a_hbm, b_hbm, c_hbm, d_hbm, x_hbm, out_hbm,
                 a_v, b_v, c_v, d_v, xbuf, obuf):
    wid = lax.axis_index("s") * NUM_CORES + lax.axis_index("c")
    base = wid * PER_W

    pltpu.sync_copy(a_hbm, a_v)
    pltpu.sync_copy(b_hbm, b_v)
    pltpu.sync_copy(c_hbm, c_v)
    pltpu.sync_copy(d_hbm, d_v)

    scale = jnp.float32(NUM_KNOTS - 1)

    def block(blk, _):
        off = base + blk * BLK
        pltpu.sync_copy(x_hbm.at[pl.ds(off, BLK)], xbuf)

        @plsc.parallel_loop(0, BLK, LANES, unroll=8)
        def vec(v):
            xv = xbuf[pl.ds(v, LANES)]
            xv = jnp.minimum(jnp.maximum(xv, jnp.float32(0.0)), jnp.float32(1.0))
            tf = xv * scale
            i = jnp.minimum(tf.astype(jnp.int32), NUM_KNOTS - 2)
            t = tf - i.astype(jnp.float32)
            av = plsc.load_gather(a_v, [i])
            bv = plsc.load_gather(b_v, [i])
            cv = plsc.load_gather(c_v, [i])
            dv = plsc.load_gather(d_v, [i])
            r = ((dv * t + cv) * t + bv) * t + av
            obuf[pl.ds(v, LANES)] = r

        pltpu.sync_copy(obuf, out_hbm.at[pl.ds(off, BLK)])
        return _

    lax.fori_loop(0, NBLK, block, None)


@functools.partial(jax.jit, static_argnames=())
def kernel(x, values):
    v = values.astype(jnp.float32)
    pm1 = jnp.concatenate([v[:1], v[:-1]])
    pp1 = jnp.concatenate([v[1:], v[-1:]])
    pp2 = jnp.concatenate([v[2:], v[-1:], v[-1:]])
    at = v
    bt = 0.5 * (pp1 - pm1)
    ct = 0.5 * (2.0 * pm1 - 5.0 * v + 4.0 * pp1 - pp2)
    dt = 0.5 * (-pm1 + 3.0 * v - 3.0 * pp1 + pp2)

    xf = x.reshape(-1)

    run = pl.kernel(
        _spline_body,
        out_type=jax.ShapeDtypeStruct((N_TOTAL,), jnp.float32),
        mesh=plsc.VectorSubcoreMesh(
            core_axis_name="c", subcore_axis_name="s",
            num_cores=NUM_CORES, num_subcores=NUM_SUBCORES),
        compiler_params=pltpu.CompilerParams(needs_layout_passes=False),
        scratch_types=[
            pltpu.VMEM((NUM_KNOTS,), jnp.float32),
            pltpu.VMEM((NUM_KNOTS,), jnp.float32),
            pltpu.VMEM((NUM_KNOTS,), jnp.float32),
            pltpu.VMEM((NUM_KNOTS,), jnp.float32),
            pltpu.VMEM((BLK,), jnp.float32),
            pltpu.VMEM((BLK,), jnp.float32),
        ],
    )
    out = run(at, bt, ct, dt, xf)
    return out.reshape(x.shape)


# speedup vs baseline: 3.1175x; 1.4948x over previous
import functools

import jax
import jax.numpy as jnp
from jax import lax
from jax.experimental import pallas as pl
from jax.experimental.pallas import tpu as pltpu
from jax.experimental.pallas import tpu_sc as plsc

NUM_KNOTS = 64
LANES = 16
NUM_CORES = 2
NUM_SUBCORES = 16
NW = NUM_CORES * NUM_SUBCORES

ROWS, COLS = 16384, 200
ROWS_PER_W = ROWS // NW
BLK_R = 128
NBLK = ROWS_PER_W // BLK_R
COL_OFFS = tuple(range(0, COLS - LANES + 1, LANES)) + (COLS - LANES,)


def _spline_body(a_hbm, b_hbm, c_hbm, d_hbm, x_hbm, out_hbm,
                 a_v, b_v, c_v, d_v, xbuf, obuf):
    wid = lax.axis_index("s") * NUM_CORES + lax.axis_index("c")
    row0 = wid * ROWS_PER_W

    pltpu.sync_copy(a_hbm, a_v)
    pltpu.sync_copy(b_hbm, b_v)
    pltpu.sync_copy(c_hbm, c_v)
    pltpu.sync_copy(d_hbm, d_v)

    scale = jnp.float32(NUM_KNOTS - 1)
    lane = jnp.arange(LANES, dtype=jnp.int32)

    def block(blk, _):
        r0 = row0 + blk * BLK_R
        pltpu.sync_copy(x_hbm.at[pl.ds(r0, BLK_R)], xbuf)

        @plsc.parallel_loop(0, BLK_R, 1, unroll=2)
        def row(r):
            for c in COL_OFFS:
                xv = xbuf[r, pl.ds(c, LANES)]
                xv = jnp.minimum(jnp.maximum(xv, jnp.float32(0.0)),
                                 jnp.float32(1.0))
                tf = xv * scale
                i = jnp.minimum(tf.astype(jnp.int32), NUM_KNOTS - 2)
                t = tf - i.astype(jnp.float32)
                j = i * LANES + lane
                av = plsc.load_gather(a_v, [j])
                bv = plsc.load_gather(b_v, [j])
                cv = plsc.load_gather(c_v, [j])
                dv = plsc.load_gather(d_v, [j])
                obuf[r, pl.ds(c, LANES)] = ((dv * t + cv) * t + bv) * t + av

        pltpu.sync_copy(obuf, out_hbm.at[pl.ds(r0, BLK_R)])
        return _

    lax.fori_loop(0, NBLK, block, None)


@jax.jit
def kernel(x, values):
    v = values.astype(jnp.float32)
    pm1 = jnp.concatenate([v[:1], v[:-1]])
    pp1 = jnp.concatenate([v[1:], v[-1:]])
    pp2 = jnp.concatenate([v[2:], v[-1:], v[-1:]])
    at = v
    bt = 0.5 * (pp1 - pm1)
    ct = 0.5 * (2.0 * pm1 - 5.0 * v + 4.0 * pp1 - pp2)
    dt = 0.5 * (-pm1 + 3.0 * v - 3.0 * pp1 + pp2)
    at, bt, ct, dt = (jnp.repeat(z, LANES) for z in (at, bt, ct, dt))

    run = pl.kernel(
        _spline_body,
        out_type=jax.ShapeDtypeStruct((ROWS, COLS), jnp.float32),
        mesh=plsc.VectorSubcoreMesh(
            core_axis_name="c", subcore_axis_name="s",
            num_cores=NUM_CORES, num_subcores=NUM_SUBCORES),
        compiler_params=pltpu.CompilerParams(needs_layout_passes=False),
        scratch_types=[
            pltpu.VMEM((NUM_KNOTS * LANES,), jnp.float32),
            pltpu.VMEM((NUM_KNOTS * LANES,), jnp.float32),
            pltpu.VMEM((NUM_KNOTS * LANES,), jnp.float32),
            pltpu.VMEM((NUM_KNOTS * LANES,), jnp.float32),
            pltpu.VMEM((BLK_R, COLS), jnp.float32),
            pltpu.VMEM((BLK_R, COLS), jnp.float32),
        ],
    )
    return run(at, bt, ct, dt, x)


# speedup vs baseline: 3.4837x; 1.1175x over previous
import functools

import jax
import jax.numpy as jnp
from jax import lax
from jax.experimental import pallas as pl
from jax.experimental.pallas import tpu as pltpu
from jax.experimental.pallas import tpu_sc as plsc

NUM_KNOTS = 64
LANES = 16
NUM_CORES = 2
NUM_SUBCORES = 16
NW = NUM_CORES * NUM_SUBCORES

ROWS, COLS = 16384, 200
ROWS_PER_W = ROWS // NW
BLK_R = 64
NBLK = ROWS_PER_W // BLK_R
COL_OFFS = tuple(range(0, COLS - LANES + 1, LANES)) + (COLS - LANES,)


def _spline_body(a_hbm, b_hbm, c_hbm, d_hbm, x_hbm, out_hbm,
                 a_v, b_v, c_v, d_v, xbuf, obuf, in_sems, out_sems):
    wid = lax.axis_index("s") * NUM_CORES + lax.axis_index("c")
    row0 = wid * ROWS_PER_W

    def in_copy(k, b):
        return pltpu.async_copy(
            x_hbm.at[pl.ds(row0 + k * BLK_R, BLK_R)],
            xbuf.at[b],
            in_sems.at[b])

    def out_copy(k, b):
        return pltpu.async_copy(
            obuf.at[b],
            out_hbm.at[pl.ds(row0 + k * BLK_R, BLK_R)],
            out_sems.at[b])

    def wait_in(b):
        pltpu.make_async_copy(
            x_hbm.at[pl.ds(row0, BLK_R)],
            xbuf.at[b],
            in_sems.at[b]).wait()

    def wait_out(b):
        pltpu.make_async_copy(
            obuf.at[b],
            out_hbm.at[pl.ds(row0, BLK_R)],
            out_sems.at[b]).wait()

    in_copy(0, 0)
    in_copy(1, 1)

    pltpu.sync_copy(a_hbm, a_v)
    pltpu.sync_copy(b_hbm, b_v)
    pltpu.sync_copy(c_hbm, c_v)
    pltpu.sync_copy(d_hbm, d_v)

    scale = jnp.float32(NUM_KNOTS - 1)
    lane = jnp.arange(LANES, dtype=jnp.int32)

    def block(k, _):
        b = lax.rem(k, 2)
        wait_in(b)

        @pl.when(k >= 2)
        def _wait_out():
            wait_out(b)

        @plsc.parallel_loop(0, BLK_R, 1, unroll=2)
        def row(r):
            for c in COL_OFFS:
                xv = xbuf[b, r, pl.ds(c, LANES)]
                xv = jnp.minimum(jnp.maximum(xv, jnp.float32(0.0)),
                                 jnp.float32(1.0))
                tf = xv * scale
                i = tf.astype(jnp.int32)
                i = jnp.minimum(jnp.maximum(i, 0), NUM_KNOTS - 2)
                t = tf - i.astype(jnp.float32)
                j = i * LANES + lane
                av = plsc.load_gather(a_v, [j])
                bv = plsc.load_gather(b_v, [j])
                cv = plsc.load_gather(c_v, [j])
                dv = plsc.load_gather(d_v, [j])
                obuf[b, r, pl.ds(c, LANES)] = \
                    ((dv * t + cv) * t + bv) * t + av

        out_copy(k, b)

        @pl.when(k + 2 < NBLK)
        def _next_in():
            in_copy(k + 2, b)

        return _

    lax.fori_loop(0, NBLK, block, None)
    wait_out(0)
    wait_out(1)


@jax.jit
def kernel(x, values):
    v = values.astype(jnp.float32)
    pm1 = jnp.concatenate([v[:1], v[:-1]])
    pp1 = jnp.concatenate([v[1:], v[-1:]])
    pp2 = jnp.concatenate([v[2:], v[-1:], v[-1:]])
    at = v
    bt = 0.5 * (pp1 - pm1)
    ct = 0.5 * (2.0 * pm1 - 5.0 * v + 4.0 * pp1 - pp2)
    dt = 0.5 * (-pm1 + 3.0 * v - 3.0 * pp1 + pp2)
    at, bt, ct, dt = (jnp.repeat(z, LANES) for z in (at, bt, ct, dt))

    run = pl.kernel(
        _spline_body,
        out_type=jax.ShapeDtypeStruct((ROWS, COLS), jnp.float32),
        mesh=plsc.VectorSubcoreMesh(
            core_axis_name="c", subcore_axis_name="s",
            num_cores=NUM_CORES, num_subcores=NUM_SUBCORES),
        compiler_params=pltpu.CompilerParams(needs_layout_passes=False),
        scratch_types=[
            pltpu.VMEM((NUM_KNOTS * LANES,), jnp.float32),
            pltpu.VMEM((NUM_KNOTS * LANES,), jnp.float32),
            pltpu.VMEM((NUM_KNOTS * LANES,), jnp.float32),
            pltpu.VMEM((NUM_KNOTS * LANES,), jnp.float32),
            pltpu.VMEM((2, BLK_R, COLS), jnp.float32),
            pltpu.VMEM((2, BLK_R, COLS), jnp.float32),
            pltpu.SemaphoreType.DMA((2,)),
            pltpu.SemaphoreType.DMA((2,)),
        ],
    )
    return run(at, bt, ct, dt, x)


# speedup vs baseline: 3.4866x; 1.0008x over previous
import functools

import jax
import jax.numpy as jnp
from jax import lax
from jax.experimental import pallas as pl
from jax.experimental.pallas import tpu as pltpu
from jax.experimental.pallas import tpu_sc as plsc

NUM_KNOTS = 64
LANES = 16
NUM_CORES = 2
NUM_SUBCORES = 16
NW = NUM_CORES * NUM_SUBCORES

ROWS, COLS = 16384, 200
ROWS_PER_W = ROWS // NW
BLK_R = 64
NBLK = ROWS_PER_W // BLK_R
COL_OFFS = tuple(range(0, COLS - LANES + 1, LANES)) + (COLS - LANES,)


def _spline_body(a_hbm, b_hbm, c_hbm, d_hbm, x_hbm, out_hbm,
                 a_v, b_v, c_v, d_v, xbuf, obuf, in_sems, out_sems):
    wid = lax.axis_index("s") * NUM_CORES + lax.axis_index("c")
    row0 = wid * ROWS_PER_W

    def in_copy(k, b):
        return pltpu.async_copy(
            x_hbm.at[pl.ds(row0 + k * BLK_R, BLK_R)],
            xbuf.at[b],
            in_sems.at[b])

    def out_copy(k, b):
        return pltpu.async_copy(
            obuf.at[b],
            out_hbm.at[pl.ds(row0 + k * BLK_R, BLK_R)],
            out_sems.at[b])

    def wait_in(b):
        pltpu.make_async_copy(
            x_hbm.at[pl.ds(row0, BLK_R)],
            xbuf.at[b],
            in_sems.at[b]).wait()

    def wait_out(b):
        pltpu.make_async_copy(
            obuf.at[b],
            out_hbm.at[pl.ds(row0, BLK_R)],
            out_sems.at[b]).wait()

    in_copy(0, 0)
    in_copy(1, 1)

    pltpu.sync_copy(a_hbm, a_v)
    pltpu.sync_copy(b_hbm, b_v)
    pltpu.sync_copy(c_hbm, c_v)
    pltpu.sync_copy(d_hbm, d_v)

    scale = jnp.float32(NUM_KNOTS - 1)
    lane = jnp.arange(LANES, dtype=jnp.int32)

    def block(k, _):
        b = lax.rem(k, 2)
        wait_in(b)

        @pl.when(k >= 2)
        def _wait_out():
            wait_out(b)

        @plsc.parallel_loop(0, BLK_R, 1, unroll=2)
        def row(r):
            for c in COL_OFFS:
                xv = xbuf[b, r, pl.ds(c, LANES)]
                xv = jnp.minimum(jnp.maximum(xv, jnp.float32(0.0)),
                                 jnp.float32(1.0))
                tf = xv * scale
                i = tf.astype(jnp.int32)
                i = jnp.minimum(jnp.maximum(i, 0), NUM_KNOTS - 2)
                t = tf - i.astype(jnp.float32)
                j = i * LANES + lane
                av = plsc.load_gather(a_v, [j])
                bv = plsc.load_gather(b_v, [j])
                cv = plsc.load_gather(c_v, [j])
                dv = plsc.load_gather(d_v, [j])
                obuf[b, r, pl.ds(c, LANES)] = \
                    ((dv * t + cv) * t + bv) * t + av

        out_copy(k, b)

        @pl.when(k + 2 < NBLK)
        def _next_in():
            in_copy(k + 2, b)

        return _

    lax.fori_loop(0, NBLK, block, None)
    wait_out(0)
    wait_out(1)


@jax.jit
def kernel(x, values):
    v = values.astype(jnp.float32)
    pm1 = jnp.concatenate([v[:1], v[:-1]])
    pp1 = jnp.concatenate([v[1:], v[-1:]])
    pp2 = jnp.concatenate([v[2:], v[-1:], v[-1:]])
    at = v
    bt = 0.5 * (pp1 - pm1)
    ct = 0.5 * (2.0 * pm1 - 5.0 * v + 4.0 * pp1 - pp2)
    dt = 0.5 * (-pm1 + 3.0 * v - 3.0 * pp1 + pp2)
    at, bt, ct, dt = (jnp.repeat(z, LANES) for z in (at, bt, ct, dt))

    run = pl.kernel(
        _spline_body,
        out_type=jax.ShapeDtypeStruct((ROWS, COLS), jnp.float32),
        mesh=plsc.VectorSubcoreMesh(
            core_axis_name="c", subcore_axis_name="s",
            num_cores=NUM_CORES, num_subcores=NUM_SUBCORES),
        compiler_params=pltpu.CompilerParams(
            needs_layout_passes=False, skip_device_barrier=True),
        scratch_types=[
            pltpu.VMEM((NUM_KNOTS * LANES,), jnp.float32),
            pltpu.VMEM((NUM_KNOTS * LANES,), jnp.float32),
            pltpu.VMEM((NUM_KNOTS * LANES,), jnp.float32),
            pltpu.VMEM((NUM_KNOTS * LANES,), jnp.float32),
            pltpu.VMEM((2, BLK_R, COLS), jnp.float32),
            pltpu.VMEM((2, BLK_R, COLS), jnp.float32),
            pltpu.SemaphoreType.DMA((2,)),
            pltpu.SemaphoreType.DMA((2,)),
        ],
    )
    return run(at, bt, ct, dt, x)


# speedup vs baseline: 3.6204x; 1.0384x over previous
import functools

import jax
import jax.numpy as jnp
from jax import lax
from jax.experimental import pallas as pl
from jax.experimental.pallas import tpu as pltpu
from jax.experimental.pallas import tpu_sc as plsc

NUM_KNOTS = 64
LANES = 16
NUM_CORES = 2
NUM_SUBCORES = 16
NW = NUM_CORES * NUM_SUBCORES

ROWS, COLS = 16384, 200
ROWS_PER_W = ROWS // NW
BLK_R = 64
NBLK = ROWS_PER_W // BLK_R
COL_OFFS = tuple(range(0, COLS - LANES + 1, LANES)) + (COLS - LANES,)


def _spline_body(a_hbm, b_hbm, c_hbm, d_hbm, x_hbm, out_hbm,
                 a_v, b_v, c_v, d_v, xbuf, obuf, in_sems, out_sems):
    wid = lax.axis_index("s") * NUM_CORES + lax.axis_index("c")
    row0 = wid * ROWS_PER_W

    def in_copy(k, b):
        return pltpu.async_copy(
            x_hbm.at[pl.ds(row0 + k * BLK_R, BLK_R)],
            xbuf.at[b],
            in_sems.at[b])

    def out_copy(k, b):
        return pltpu.async_copy(
            obuf.at[b],
            out_hbm.at[pl.ds(row0 + k * BLK_R, BLK_R)],
            out_sems.at[b])

    def wait_in(b):
        pltpu.make_async_copy(
            x_hbm.at[pl.ds(row0, BLK_R)],
            xbuf.at[b],
            in_sems.at[b]).wait()

    def wait_out(b):
        pltpu.make_async_copy(
            obuf.at[b],
            out_hbm.at[pl.ds(row0, BLK_R)],
            out_sems.at[b]).wait()

    in_copy(0, 0)
    in_copy(1, 1)

    pltpu.sync_copy(a_hbm, a_v)
    pltpu.sync_copy(b_hbm, b_v)
    pltpu.sync_copy(c_hbm, c_v)
    pltpu.sync_copy(d_hbm, d_v)

    scale = jnp.float32(NUM_KNOTS - 1)
    lane = jnp.arange(LANES, dtype=jnp.int32)

    def block(k, _):
        b = lax.rem(k, 2)
        wait_in(b)

        @pl.when(k >= 2)
        def _wait_out():
            wait_out(b)

        @plsc.parallel_loop(0, BLK_R, 1, unroll=1)
        def row(r):
            for c in COL_OFFS:
                xv = xbuf[b, r, pl.ds(c, LANES)]
                xv = jnp.minimum(jnp.maximum(xv, jnp.float32(0.0)),
                                 jnp.float32(1.0))
                tf = xv * scale
                i = tf.astype(jnp.int32)
                i = jnp.minimum(jnp.maximum(i, 0), NUM_KNOTS - 2)
                t = tf - i.astype(jnp.float32)
                j = i * LANES + lane
                av = plsc.load_gather(a_v, [j])
                bv = plsc.load_gather(b_v, [j])
                cv = plsc.load_gather(c_v, [j])
                dv = plsc.load_gather(d_v, [j])
                obuf[b, r, pl.ds(c, LANES)] = \
                    ((dv * t + cv) * t + bv) * t + av

        out_copy(k, b)

        @pl.when(k + 2 < NBLK)
        def _next_in():
            in_copy(k + 2, b)

        return _

    lax.fori_loop(0, NBLK, block, None)
    wait_out(0)
    wait_out(1)


@jax.jit
def kernel(x, values):
    v = values.astype(jnp.float32)
    pm1 = jnp.concatenate([v[:1], v[:-1]])
    pp1 = jnp.concatenate([v[1:], v[-1:]])
    pp2 = jnp.concatenate([v[2:], v[-1:], v[-1:]])
    at = v
    bt = 0.5 * (pp1 - pm1)
    ct = 0.5 * (2.0 * pm1 - 5.0 * v + 4.0 * pp1 - pp2)
    dt = 0.5 * (-pm1 + 3.0 * v - 3.0 * pp1 + pp2)
    at, bt, ct, dt = (jnp.repeat(z, LANES) for z in (at, bt, ct, dt))

    run = pl.kernel(
        _spline_body,
        out_type=jax.ShapeDtypeStruct((ROWS, COLS), jnp.float32),
        mesh=plsc.VectorSubcoreMesh(
            core_axis_name="c", subcore_axis_name="s",
            num_cores=NUM_CORES, num_subcores=NUM_SUBCORES),
        compiler_params=pltpu.CompilerParams(needs_layout_passes=False),
        scratch_types=[
            pltpu.VMEM((NUM_KNOTS * LANES,), jnp.float32),
            pltpu.VMEM((NUM_KNOTS * LANES,), jnp.float32),
            pltpu.VMEM((NUM_KNOTS * LANES,), jnp.float32),
            pltpu.VMEM((NUM_KNOTS * LANES,), jnp.float32),
            pltpu.VMEM((2, BLK_R, COLS), jnp.float32),
            pltpu.VMEM((2, BLK_R, COLS), jnp.float32),
            pltpu.SemaphoreType.DMA((2,)),
            pltpu.SemaphoreType.DMA((2,)),
        ],
    )
    return run(at, bt, ct, dt, x)


# speedup vs baseline: 3.7033x; 1.0229x over previous
import functools

import jax
import jax.numpy as jnp
from jax import lax
from jax.experimental import pallas as pl
from jax.experimental.pallas import tpu as pltpu
from jax.experimental.pallas import tpu_sc as plsc

NUM_KNOTS = 64
LANES = 16
NUM_CORES = 2
NUM_SUBCORES = 16
NW = NUM_CORES * NUM_SUBCORES

ROWS, COLS = 16384, 200
ROWS_PER_W = ROWS // NW
BLK_R = 64
NBLK = ROWS_PER_W // BLK_R
COL_OFFS = tuple(range(0, COLS - LANES + 1, LANES)) + (COLS - LANES,)


def _spline_body(a_hbm, b_hbm, c_hbm, d_hbm, x_hbm, out_hbm,
                 a_v, b_v, c_v, d_v, xbuf, obuf, in_sems, out_sems):
    wid = lax.axis_index("s") * NUM_CORES + lax.axis_index("c")
    row0 = wid * ROWS_PER_W

    def in_copy(k, b):
        return pltpu.async_copy(
            x_hbm.at[pl.ds(row0 + k * BLK_R, BLK_R)],
            xbuf.at[b],
            in_sems.at[b])

    def out_copy(k, b):
        return pltpu.async_copy(
            obuf.at[b],
            out_hbm.at[pl.ds(row0 + k * BLK_R, BLK_R)],
            out_sems.at[b])

    def wait_in(b):
        pltpu.make_async_copy(
            x_hbm.at[pl.ds(row0, BLK_R)],
            xbuf.at[b],
            in_sems.at[b]).wait()

    def wait_out(b):
        pltpu.make_async_copy(
            obuf.at[b],
            out_hbm.at[pl.ds(row0, BLK_R)],
            out_sems.at[b]).wait()

    in_copy(0, 0)
    in_copy(1, 1)

    pltpu.sync_copy(a_hbm, a_v)
    pltpu.sync_copy(b_hbm, b_v)
    pltpu.sync_copy(c_hbm, c_v)
    pltpu.sync_copy(d_hbm, d_v)

    scale = jnp.float32(NUM_KNOTS - 1)
    lane = jnp.arange(LANES, dtype=jnp.int32)

    def block(k, _):
        b = lax.rem(k, 2)
        wait_in(b)

        @pl.when(k >= 2)
        def _wait_out():
            wait_out(b)

        @plsc.parallel_loop(0, BLK_R, 1, unroll=1)
        def row(r):
            for c in COL_OFFS:
                xv = xbuf[b, r, pl.ds(c, LANES)]
                tf = xv * scale
                i = tf.astype(jnp.int32)
                i = jnp.minimum(jnp.maximum(i, 0), NUM_KNOTS - 2)
                t = tf - i.astype(jnp.float32)
                j = i * LANES + lane
                av = plsc.load_gather(a_v, [j])
                bv = plsc.load_gather(b_v, [j])
                cv = plsc.load_gather(c_v, [j])
                dv = plsc.load_gather(d_v, [j])
                obuf[b, r, pl.ds(c, LANES)] = \
                    ((dv * t + cv) * t + bv) * t + av

        out_copy(k, b)

        @pl.when(k + 2 < NBLK)
        def _next_in():
            in_copy(k + 2, b)

        return _

    lax.fori_loop(0, NBLK, block, None)
    wait_out(0)
    wait_out(1)


@jax.jit
def kernel(x, values):
    v = values.astype(jnp.float32)
    pm1 = jnp.concatenate([v[:1], v[:-1]])
    pp1 = jnp.concatenate([v[1:], v[-1:]])
    pp2 = jnp.concatenate([v[2:], v[-1:], v[-1:]])
    at = v
    bt = 0.5 * (pp1 - pm1)
    ct = 0.5 * (2.0 * pm1 - 5.0 * v + 4.0 * pp1 - pp2)
    dt = 0.5 * (-pm1 + 3.0 * v - 3.0 * pp1 + pp2)
    at, bt, ct, dt = (jnp.repeat(z, LANES) for z in (at, bt, ct, dt))

    run = pl.kernel(
        _spline_body,
        out_type=jax.ShapeDtypeStruct((ROWS, COLS), jnp.float32),
        mesh=plsc.VectorSubcoreMesh(
            core_axis_name="c", subcore_axis_name="s",
            num_cores=NUM_CORES, num_subcores=NUM_SUBCORES),
        compiler_params=pltpu.CompilerParams(needs_layout_passes=False),
        scratch_types=[
            pltpu.VMEM((NUM_KNOTS * LANES,), jnp.float32),
            pltpu.VMEM((NUM_KNOTS * LANES,), jnp.float32),
            pltpu.VMEM((NUM_KNOTS * LANES,), jnp.float32),
            pltpu.VMEM((NUM_KNOTS * LANES,), jnp.float32),
            pltpu.VMEM((2, BLK_R, COLS), jnp.float32),
            pltpu.VMEM((2, BLK_R, COLS), jnp.float32),
            pltpu.SemaphoreType.DMA((2,)),
            pltpu.SemaphoreType.DMA((2,)),
        ],
    )
    return run(at, bt, ct, dt, x)


# speedup vs baseline: 3.7971x; 1.0253x over previous
import jax
import jax.numpy as jnp
from jax import lax
from jax.experimental import pallas as pl
from jax.experimental.pallas import tpu as pltpu
from jax.experimental.pallas import tpu_sc as plsc

NUM_KNOTS = 64
LANES = 16
NUM_CORES = 2
NUM_SUBCORES = 16
NW = NUM_CORES * NUM_SUBCORES
TBL = NUM_KNOTS * LANES

ROWS, COLS = 16384, 200
ROWS_PER_W = ROWS // NW
BLK_R = 64
NBLK = ROWS_PER_W // BLK_R
COL_OFFS = tuple(range(0, COLS - LANES + 1, LANES)) + (COLS - LANES,)


def _spline_body(tbl_hbm, x_hbm, out_hbm,
                 a_v, b_v, c_v, d_v, xbuf, obuf, in_sems, out_sems):
    wid = lax.axis_index("s") * NUM_CORES + lax.axis_index("c")
    row0 = wid * ROWS_PER_W

    def in_copy(k, b):
        return pltpu.async_copy(
            x_hbm.at[pl.ds(row0 + k * BLK_R, BLK_R)], xbuf.at[b],
            in_sems.at[b])

    def out_copy(k, b):
        return pltpu.async_copy(
            obuf.at[b], out_hbm.at[pl.ds(row0 + k * BLK_R, BLK_R)],
            out_sems.at[b])

    def wait_in(b):
        pltpu.make_async_copy(
            x_hbm.at[pl.ds(row0, BLK_R)], xbuf.at[b], in_sems.at[b]).wait()

    def wait_out(b):
        pltpu.make_async_copy(
            obuf.at[b], out_hbm.at[pl.ds(row0, BLK_R)],
            out_sems.at[b]).wait()

    in_copy(0, 0)
    in_copy(1, 1)

    pltpu.sync_copy(tbl_hbm.at[0], a_v)
    pltpu.sync_copy(tbl_hbm.at[1], b_v)
    pltpu.sync_copy(tbl_hbm.at[2], c_v)
    pltpu.sync_copy(tbl_hbm.at[3], d_v)

    scale = jnp.float32(NUM_KNOTS - 1)
    lane = jnp.arange(LANES, dtype=jnp.int32)

    def block(k, _):
        b = lax.rem(k, 2)
        wait_in(b)

        @pl.when(k >= 2)
        def _wait_out():
            wait_out(b)

        @plsc.parallel_loop(0, BLK_R, 1, unroll=1)
        def row(r):
            for c in COL_OFFS:
                xv = xbuf[b, r, pl.ds(c, LANES)]
                tf = xv * scale
                i = tf.astype(jnp.int32)
                i = jnp.minimum(jnp.maximum(i, 0), NUM_KNOTS - 2)
                t = tf - i.astype(jnp.float32)
                j = i * LANES + lane
                av = plsc.load_gather(a_v, [j])
                bv = plsc.load_gather(b_v, [j])
                cv = plsc.load_gather(c_v, [j])
                dv = plsc.load_gather(d_v, [j])
                obuf[b, r, pl.ds(c, LANES)] = \
                    ((dv * t + cv) * t + bv) * t + av

        out_copy(k, b)

        @pl.when(k + 2 < NBLK)
        def _next_in():
            in_copy(k + 2, b)

        return _

    lax.fori_loop(0, NBLK, block, None)
    wait_out(0)
    wait_out(1)


@jax.jit
def kernel(x, values):
    v = values.astype(jnp.float32)
    pm1 = jnp.concatenate([v[:1], v[:-1]])
    pp1 = jnp.concatenate([v[1:], v[-1:]])
    pp2 = jnp.concatenate([v[2:], v[-1:], v[-1:]])
    at = v
    bt = 0.5 * (pp1 - pm1)
    ct = 0.5 * (2.0 * pm1 - 5.0 * v + 4.0 * pp1 - pp2)
    dt = 0.5 * (-pm1 + 3.0 * v - 3.0 * pp1 + pp2)
    tbl = jnp.stack([jnp.repeat(z, LANES) for z in (at, bt, ct, dt)])

    run = pl.kernel(
        _spline_body,
        out_type=jax.ShapeDtypeStruct((ROWS, COLS), jnp.float32),
        mesh=plsc.VectorSubcoreMesh(
            core_axis_name="c", subcore_axis_name="s",
            num_cores=NUM_CORES, num_subcores=NUM_SUBCORES),
        compiler_params=pltpu.CompilerParams(needs_layout_passes=False),
        scratch_types=[
            pltpu.VMEM((TBL,), jnp.float32),
            pltpu.VMEM((TBL,), jnp.float32),
            pltpu.VMEM((TBL,), jnp.float32),
            pltpu.VMEM((TBL,), jnp.float32),
            pltpu.VMEM((2, BLK_R, COLS), jnp.float32),
            pltpu.VMEM((2, BLK_R, COLS), jnp.float32),
            pltpu.SemaphoreType.DMA((2,)),
            pltpu.SemaphoreType.DMA((2,)),
        ],
    )
    return run(tbl, x)
